# baseline (speedup 1.0000x reference)
import functools

import jax
import jax.numpy as jnp
from jax import lax
from jax.experimental import pallas as pl
from jax.experimental.pallas import tpu as pltpu
from jax.experimental.pallas import tpu_sc as plsc

D_MODEL = 1024
T_TOTAL = 8192
B_BATCH = 4
OFFSET = 20

NUM_CORES = 2
NUM_SUBCORES = 16
NW = NUM_CORES * NUM_SUBCORES
TS_PER_W = T_TOTAL // NW
CH = 16
NCHUNK = TS_PER_W // CH
LANES = 16
DCH = D_MODEL // LANES


def _pe_add_body(x_hbm, t_hbm, pe_hbm, out_hbm, idx_v, xb, peb, semx, semp):
    wid = lax.axis_index("s") * NUM_CORES + lax.axis_index("c")
    base = wid * TS_PER_W

    pltpu.sync_copy(t_hbm.at[pl.ds(base, TS_PER_W)], idx_v)

    def add_off(i, carry):
        sl = pl.ds(i * LANES, LANES)
        idx_v[sl] = idx_v[sl] + OFFSET
        return carry

    lax.fori_loop(0, TS_PER_W // LANES, add_off, 0, unroll=4)

    def chunk_body(c, carry):
        tbase = base + c * CH
        cpx = pltpu.async_copy(x_hbm.at[pl.ds(tbase, CH)], xb, semx)
        cpp = pltpu.async_copy(pe_hbm.at[idx_v.at[pl.ds(c * CH, CH)]], peb, semp)
        cpx.wait()
        cpp.wait()
        for t in range(CH):
            for b in range(B_BATCH):
                def body(k, carry2, t=t, b=b):
                    sl = pl.ds(k * LANES, LANES)
                    xb[t, b, sl] = xb[t, b, sl] + peb[t, sl]
                    return carry2

                lax.fori_loop(0, DCH, body, 0, unroll=8)
        pltpu.sync_copy(xb, out_hbm.at[pl.ds(tbase, CH)])
        return carry

    lax.fori_loop(0, NCHUNK, chunk_body, 0)


_pe_add = functools.partial(
    pl.kernel,
    out_type=jax.ShapeDtypeStruct((T_TOTAL, B_BATCH, D_MODEL), jnp.float32),
    mesh=plsc.VectorSubcoreMesh(core_axis_name="c", subcore_axis_name="s"),
    scratch_types=[
        pltpu.VMEM((TS_PER_W,), jnp.int32),
        pltpu.VMEM((CH, B_BATCH, D_MODEL), jnp.float32),
        pltpu.VMEM((CH, D_MODEL), jnp.float32),
        pltpu.SemaphoreType.DMA,
        pltpu.SemaphoreType.DMA,
    ],
)(_pe_add_body)


def kernel(x, time_tensor, pe):
    return _pe_add(x, time_tensor.astype(jnp.int32), pe)


# speedup vs baseline: 2.4989x; 2.4989x over previous
import functools

import jax
import jax.numpy as jnp
from jax import lax
from jax.experimental import pallas as pl
from jax.experimental.pallas import tpu as pltpu
from jax.experimental.pallas import tpu_sc as plsc

D_MODEL = 1024
T_TOTAL = 8192
B_BATCH = 4
OFFSET = 20

NUM_CORES = 2
NUM_SUBCORES = 16
NW = NUM_CORES * NUM_SUBCORES
TS_PER_W = T_TOTAL // NW
CH = 4
NCHUNK = TS_PER_W // CH
NSLOT = 4
NGRP = NCHUNK // NSLOT
LANES = 16
DCH = D_MODEL // LANES


def _pe_add_body(x_hbm, t_hbm, pe_hbm, out_hbm, *refs):
    xbs = refs[0:NSLOT]
    pebs = refs[NSLOT:2 * NSLOT]
    idx_v = refs[2 * NSLOT]
    sin = refs[2 * NSLOT + 1:2 * NSLOT + 1 + NSLOT]
    sout = refs[2 * NSLOT + 1 + NSLOT:2 * NSLOT + 1 + 2 * NSLOT]

    wid = lax.axis_index("s") * NUM_CORES + lax.axis_index("c")
    base = wid * TS_PER_W

    pltpu.sync_copy(t_hbm.at[wid], idx_v)

    def start_in(c, s):
        tbase = base + c * CH
        pltpu.async_copy(x_hbm.at[pl.ds(tbase, CH)], xbs[s], sin[s])
        pltpu.async_copy(pe_hbm.at[idx_v.at[c]], pebs[s], sin[s])

    def wait_in(c, s):
        tbase = base + c * CH
        pltpu.make_async_copy(x_hbm.at[pl.ds(tbase, CH)], xbs[s], sin[s]).wait()
        pltpu.make_async_copy(pe_hbm.at[idx_v.at[c]], pebs[s], sin[s]).wait()

    def start_out(c, s):
        tbase = base + c * CH
        pltpu.async_copy(xbs[s], out_hbm.at[pl.ds(tbase, CH)], sout[s])

    def wait_out(c, s):
        tbase = base + c * CH
        pltpu.make_async_copy(xbs[s], out_hbm.at[pl.ds(tbase, CH)], sout[s]).wait()

    def compute(s):
        xb, peb = xbs[s], pebs[s]
        for t in range(CH):
            def body(k, carry, t=t):
                sl = pl.ds(k * LANES, LANES)
                pv = peb[t, sl]
                for b in range(B_BATCH):
                    xb[t, b, sl] = xb[t, b, sl] + pv
                return carry

            lax.fori_loop(0, DCH, body, 0, unroll=8)

    for s in range(NSLOT):
        start_in(s, s)

    def group_body(g, carry):
        c0 = g * NSLOT
        for s in range(NSLOT):
            c = c0 + s
            wait_in(c, s)
            compute(s)
            start_out(c, s)

            @pl.when(g < NGRP - 1)
            def _(c=c, s=s):
                wait_out(c, s)
                start_in(c + NSLOT, s)

        return carry

    lax.fori_loop(0, NGRP, group_body, 0)

    for s in range(NSLOT):
        wait_out(NCHUNK - NSLOT + s, s)


_pe_add = functools.partial(
    pl.kernel,
    out_type=jax.ShapeDtypeStruct((T_TOTAL, B_BATCH, D_MODEL), jnp.float32),
    mesh=plsc.VectorSubcoreMesh(core_axis_name="c", subcore_axis_name="s"),
    scratch_types=(
        [pltpu.VMEM((CH, B_BATCH, D_MODEL), jnp.float32) for _ in range(NSLOT)]
        + [pltpu.VMEM((CH, D_MODEL), jnp.float32) for _ in range(NSLOT)]
        + [pltpu.VMEM((NCHUNK, CH), jnp.int32)]
        + [pltpu.SemaphoreType.DMA for _ in range(2 * NSLOT)]
    ),
)(_pe_add_body)


def kernel(x, time_tensor, pe):
    idx = (time_tensor.astype(jnp.int32) + OFFSET).reshape(NW, NCHUNK, CH)
    return _pe_add(x, idx, pe)
